# Initial kernel scaffold; baseline (speedup 1.0000x reference)
import jax
import jax.numpy as jnp
from jax.experimental import pallas as pl


def kernel(h, edge_index, edge_attr, flags, edge_mask, W1, b1, W2, b2, Wn1, bn1, Wn2, bn2, Wa, ba):
    raise NotImplementedError("write your pallas kernel here")


# speedup vs baseline: 1.9800x; 1.9800x over previous
import functools

import jax
import jax.numpy as jnp
from jax import lax
from jax.experimental import pallas as pl
from jax.experimental.pallas import tpu as pltpu
from jax.experimental.pallas import tpu_sc as plsc

NC = 2
NS = 16
NW = NC * NS
CH = 128


def _silu(x):
    return x * jax.nn.sigmoid(x)


def _node_proj_body(h_ref, w1r_ref, w1c_ref, pr_ref, pc_ref):
    h = h_ref[...]
    pr_ref[...] = jnp.dot(h, w1r_ref[...], preferred_element_type=jnp.float32)
    pc_ref[...] = jnp.dot(h, w1c_ref[...], preferred_element_type=jnp.float32)


def _make_gather(e_pad, n_nodes, d):
    n_chunks = e_pad // CH
    per_w = n_chunks // NW
    mesh = plsc.VectorSubcoreMesh(core_axis_name="c", subcore_axis_name="s")

    @functools.partial(
        pl.kernel,
        out_type=[
            jax.ShapeDtypeStruct((e_pad, d), jnp.float32),
            jax.ShapeDtypeStruct((e_pad, d), jnp.float32),
        ],
        mesh=mesh,
        scratch_types=[
            pltpu.VMEM((per_w, CH), jnp.int32),
            pltpu.VMEM((per_w, CH), jnp.int32),
            pltpu.VMEM((CH, d), jnp.float32),
            pltpu.VMEM((CH, d), jnp.float32),
            pltpu.SemaphoreType.DMA,
            pltpu.SemaphoreType.DMA,
        ],
    )
    def gather(pr_hbm, pc_hbm, rowi_hbm, coli_hbm, gr_hbm, gc_hbm,
               rowi_v, coli_v, bufr, bufc, semr, semc):
        c = lax.axis_index("c")
        s = lax.axis_index("s")
        wid = s * NC + c
        base = wid * per_w
        pltpu.sync_copy(rowi_hbm.at[pl.ds(base, per_w)], rowi_v)
        pltpu.sync_copy(coli_hbm.at[pl.ds(base, per_w)], coli_v)

        def body(j, carry):
            e0 = (base + j) * CH
            cpr = pltpu.async_copy(pr_hbm.at[rowi_v.at[j]], bufr, semr)
            cpc = pltpu.async_copy(pc_hbm.at[coli_v.at[j]], bufc, semc)
            cpr.wait()
            cpc.wait()
            pltpu.sync_copy(bufr, gr_hbm.at[pl.ds(e0, CH)])
            pltpu.sync_copy(bufc, gc_hbm.at[pl.ds(e0, CH)])
            return carry

        lax.fori_loop(0, per_w, body, 0)

    return gather


def _edge_mlp_body(gr_ref, gc_ref, ea_ref, mask_ref, w1e_ref, b1_ref,
                   w2_ref, b2_ref, wa_ref, ba_ref, out_ref):
    x = (gr_ref[...] + gc_ref[...]
         + jnp.dot(ea_ref[...], w1e_ref[...], preferred_element_type=jnp.float32)
         + b1_ref[...])
    m1 = _silu(x)
    y = jnp.dot(m1, w2_ref[...], preferred_element_type=jnp.float32) + b2_ref[...]
    m2 = _silu(y)
    att = jax.nn.sigmoid(
        jnp.dot(m2, wa_ref[...], preferred_element_type=jnp.float32) + ba_ref[...])
    out_ref[...] = m2 * (att * mask_ref[...])


def _make_scatter(e_pad, n_pad, d):
    n_chunks = e_pad // CH
    per_w = n_chunks // NW
    stripe = n_pad // NS
    mesh = plsc.VectorSubcoreMesh(core_axis_name="c", subcore_axis_name="s")

    @functools.partial(
        pl.kernel,
        out_type=jax.ShapeDtypeStruct((NC * n_pad, d), jnp.float32),
        mesh=mesh,
        scratch_types=[
            pltpu.VMEM((per_w, CH), jnp.int32),
            pltpu.VMEM((CH, d), jnp.float32),
            pltpu.VMEM_SHARED((n_pad, d), jnp.float32),
            pltpu.SemaphoreType.DMA,
        ],
    )
    def scatter(m_hbm, rowi_hbm, zero_hbm, out_hbm,
                rowi_v, mbuf, acc_sh, sem):
        c = lax.axis_index("c")
        s = lax.axis_index("s")
        wid = s * NC + c
        base = wid * per_w
        pltpu.sync_copy(zero_hbm, acc_sh.at[pl.ds(s * stripe, stripe)])
        plsc.subcore_barrier()
        pltpu.sync_copy(rowi_hbm.at[pl.ds(base, per_w)], rowi_v)

        def body(j, carry):
            e0 = (base + j) * CH
            pltpu.sync_copy(m_hbm.at[pl.ds(e0, CH)], mbuf)
            pltpu.sync_copy(mbuf, acc_sh.at[rowi_v.at[j]], add=True)
            return carry

        lax.fori_loop(0, per_w, body, 0)
        plsc.subcore_barrier()
        pltpu.sync_copy(acc_sh.at[pl.ds(s * stripe, stripe)],
                        out_hbm.at[pl.ds(c * n_pad + s * stripe, stripe)])

    return scatter


def _node_update_body(h_ref, p0_ref, p1_ref, flags_ref, wn1h_ref, wn1a_ref,
                      bn1_ref, wn2_ref, bn2_ref, out_ref):
    h = h_ref[...]
    agg = p0_ref[...] + p1_ref[...]
    t = (jnp.dot(h, wn1h_ref[...], preferred_element_type=jnp.float32)
         + jnp.dot(agg, wn1a_ref[...], preferred_element_type=jnp.float32)
         + bn1_ref[...])
    t = _silu(t)
    upd = jnp.dot(t, wn2_ref[...], preferred_element_type=jnp.float32) + bn2_ref[...]
    out_ref[...] = (h + upd) * flags_ref[...]


def kernel(h, edge_index, edge_attr, flags, edge_mask,
           W1, b1, W2, b2, Wn1, bn1, Wn2, bn2, Wa, ba):
    n, d_in = h.shape
    e = edge_index.shape[1]
    d_edge = edge_attr.shape[1]
    d_out = W1.shape[1]

    W1r, W1c, W1e = W1[:d_in], W1[d_in:2 * d_in], W1[2 * d_in:]
    Wn1h, Wn1a = Wn1[:d_in], Wn1[d_in:]
    b1r = b1.reshape(1, -1)
    b2r = b2.reshape(1, -1)
    bn1r = bn1.reshape(1, -1)
    bn2r = bn2.reshape(1, -1)
    bar = ba.reshape(1, -1)

    quant = NW * CH * 8
    e_pad = ((e + quant - 1) // quant) * quant
    n_pad = ((n + NS * 8 - 1) // (NS * 8)) * (NS * 8)
    pad = e_pad - e
    row = edge_index[0].astype(jnp.int32)
    col = edge_index[1].astype(jnp.int32)
    row_p = jnp.pad(row, (0, pad)).reshape(e_pad // CH, CH)
    col_p = jnp.pad(col, (0, pad)).reshape(e_pad // CH, CH)
    ea_p = jnp.pad(edge_attr, ((0, pad), (0, 0)))
    mask_p = jnp.pad(edge_mask, ((0, pad), (0, 0)))
    zeros_stripe = jnp.zeros((n_pad // NS, d_out), jnp.float32)

    pr, pc = pl.pallas_call(
        _node_proj_body,
        out_shape=[
            jax.ShapeDtypeStruct((n, d_out), jnp.float32),
            jax.ShapeDtypeStruct((n, d_out), jnp.float32),
        ],
    )(h, W1r, W1c)

    gr, gc = _make_gather(e_pad, n, d_out)(pr, pc, row_p, col_p)

    eb = 4096
    grid = (e_pad // eb,)
    mat = lambda r, c_: pl.BlockSpec((r, c_), lambda i: (0, 0))
    msg = pl.pallas_call(
        _edge_mlp_body,
        grid=grid,
        in_specs=[
            pl.BlockSpec((eb, d_out), lambda i: (i, 0)),
            pl.BlockSpec((eb, d_out), lambda i: (i, 0)),
            pl.BlockSpec((eb, d_edge), lambda i: (i, 0)),
            pl.BlockSpec((eb, 1), lambda i: (i, 0)),
            mat(d_edge, d_out),
            mat(1, d_out),
            mat(d_out, d_out),
            mat(1, d_out),
            mat(d_out, 1),
            mat(1, 1),
        ],
        out_specs=pl.BlockSpec((eb, d_out), lambda i: (i, 0)),
        out_shape=jax.ShapeDtypeStruct((e_pad, d_out), jnp.float32),
    )(gr, gc, ea_p, mask_p, W1e, b1r, W2, b2r, Wa, bar)

    partials = _make_scatter(e_pad, n_pad, d_out)(msg, row_p, zeros_stripe)

    h_out = pl.pallas_call(
        _node_update_body,
        out_shape=jax.ShapeDtypeStruct((n, d_out), jnp.float32),
    )(h, partials[:n], partials[n_pad:n_pad + n], flags, Wn1h, Wn1a, bn1r, Wn2, bn2r)

    return h_out
